# baseline (speedup 1.0000x reference)
import jax
import jax.numpy as jnp
from jax.experimental import pallas as pl
from jax.experimental.pallas import tpu as pltpu

_S = 4096


def _embed_kernel(idx_ref, tab_ref, out_ref):
    idx = idx_ref[0, 0, :]
    f = idx.astype(jnp.float32)
    t0 = tab_ref[0, :]
    d = tab_ref[1, :] - t0
    out_ref[...] = (t0[None, :] + f[:, None] * d[None, :])[None, ...]


def kernel(inputs, table):
    B, L = inputs.shape
    H = table.shape[1]
    n = (B * L) // _S
    idx3 = inputs.reshape(n, 1, _S)
    out = pl.pallas_call(
        _embed_kernel,
        grid=(n,),
        in_specs=[
            pl.BlockSpec((1, 1, _S), lambda g: (g, 0, 0)),
            pl.BlockSpec((2, H), lambda g: (0, 0)),
        ],
        out_specs=pl.BlockSpec((1, _S, H), lambda g: (g, 0, 0)),
        out_shape=jax.ShapeDtypeStruct((n, _S, H), jnp.float32),
        compiler_params=pltpu.CompilerParams(
            dimension_semantics=("parallel",),
        ),
    )(idx3, table)
    return out.reshape(B, L, H)


# speedup vs baseline: 1.0523x; 1.0523x over previous
import jax
import jax.numpy as jnp
from jax.experimental import pallas as pl
from jax.experimental.pallas import tpu as pltpu

_S = 1024


def _embed_kernel(idx_ref, tab_ref, out_ref):
    idx = idx_ref[0, 0, :]
    f = idx.astype(jnp.float32)
    t0 = tab_ref[0, :]
    d = tab_ref[1, :] - t0
    out_ref[...] = (t0[None, :] + f[:, None] * d[None, :])[None, ...]


def kernel(inputs, table):
    B, L = inputs.shape
    H = table.shape[1]
    n = (B * L) // _S
    idx3 = inputs.reshape(n, 1, _S)
    out = pl.pallas_call(
        _embed_kernel,
        grid=(n,),
        in_specs=[
            pl.BlockSpec((1, 1, _S), lambda g: (g, 0, 0)),
            pl.BlockSpec((2, H), lambda g: (0, 0)),
        ],
        out_specs=pl.BlockSpec((1, _S, H), lambda g: (g, 0, 0)),
        out_shape=jax.ShapeDtypeStruct((n, _S, H), jnp.float32),
        compiler_params=pltpu.CompilerParams(
            dimension_semantics=("parallel",),
        ),
    )(idx3, table)
    return out.reshape(B, L, H)
